# baseline (speedup 1.0000x reference)
import functools

import jax
import jax.numpy as jnp
from jax import lax
from jax.experimental import pallas as pl
from jax.experimental.pallas import tpu as pltpu
from jax.experimental.pallas import tpu_sc as plsc

H = 768
NW = 32
CHUNK = 64
GROUP = 16


def _rsqrt16(x):
    i = lax.bitcast_convert_type(x, jnp.int32)
    i = jnp.int32(0x5F3759DF) - lax.shift_right_logical(i, 1)
    y = lax.bitcast_convert_type(i, jnp.float32)
    for _ in range(3):
        y = y * (1.5 - 0.5 * x * y * y)
    return y


def _make_sc_call(n_tokens, seq, vocab, types):
    per_w = n_tokens // NW
    n_chunks = per_w // CHUNK
    mesh = plsc.VectorSubcoreMesh(core_axis_name="c", subcore_axis_name="s")

    @functools.partial(
        pl.kernel,
        out_type=jax.ShapeDtypeStruct((n_tokens, H), jnp.float32),
        mesh=mesh,
        compiler_params=pltpu.CompilerParams(
            use_tc_tiling_on_sc=False, needs_layout_passes=False),
        scratch_types=[
            pltpu.VMEM((CHUNK,), jnp.int32),
            pltpu.VMEM((CHUNK,), jnp.int32),
            pltpu.VMEM((CHUNK, H), jnp.float32),
            pltpu.VMEM((CHUNK, H), jnp.float32),
            pltpu.VMEM((types, H), jnp.float32),
            pltpu.VMEM((H,), jnp.float32),
            pltpu.VMEM((H,), jnp.float32),
            pltpu.SemaphoreType.DMA,
        ],
    )
    def sc_call(tok_hbm, tt_hbm, wemb_hbm, pemb_hbm, temb_hbm, g_hbm, b_hbm,
                out_hbm, idx_v, tt_v, xbuf, pbuf, tybuf, gbuf, bbuf, sem):
        wid = lax.axis_index("s") * 2 + lax.axis_index("c")
        base = wid * per_w
        p0 = lax.rem(base, seq)

        pltpu.sync_copy(temb_hbm, tybuf)
        pltpu.sync_copy(g_hbm, gbuf)
        pltpu.sync_copy(b_hbm, bbuf)

        lane = lax.iota(jnp.int32, 16)
        zero = jnp.zeros((16,), jnp.float32)
        zeroi = jnp.zeros((16,), jnp.int32)

        for c in range(n_chunks):
            tb = base + c * CHUNK
            pltpu.sync_copy(tok_hbm.at[pl.ds(tb, CHUNK)], idx_v)
            pltpu.sync_copy(tt_hbm.at[pl.ds(tb, CHUNK)], tt_v)
            pltpu.async_copy(wemb_hbm.at[idx_v], xbuf, sem).wait()
            pltpu.sync_copy(pemb_hbm.at[pl.ds(p0 + c * CHUNK, CHUNK)], pbuf)

            for g in range(CHUNK // GROUP):
                t0 = g * GROUP
                ttv = tt_v[pl.ds(t0, GROUP)]
                xrow = lane + t0

                def pass1(h, carry):
                    acc, accsq, hv = carry
                    w = plsc.load_gather(xbuf, [xrow, hv])
                    p = plsc.load_gather(pbuf, [xrow, hv])
                    ty = plsc.load_gather(tybuf, [ttv, hv])
                    x = w + p + ty
                    plsc.store_scatter(xbuf, [xrow, hv], x)
                    return acc + x, accsq + x * x, hv + 1

                acc, accsq, _ = lax.fori_loop(
                    0, H, pass1, (zero, zero, zeroi), unroll=8)

                mean = acc * (1.0 / H)
                var = accsq * (1.0 / H) - mean * mean
                inv = _rsqrt16(var + 1e-12)

                def pass2(h, carry):
                    (hv,) = carry
                    x = plsc.load_gather(xbuf, [xrow, hv])
                    gm = plsc.load_gather(gbuf, [hv])
                    bt = plsc.load_gather(bbuf, [hv])
                    y = (x - mean) * inv * gm + bt
                    plsc.store_scatter(xbuf, [xrow, hv], y)
                    return (hv + 1,)

                lax.fori_loop(0, H, pass2, (zeroi,), unroll=8)

            pltpu.sync_copy(xbuf, out_hbm.at[pl.ds(tb, CHUNK)])

    return sc_call


def kernel(token_ids, token_type_ids, word_emb, pos_emb, type_emb, gamma, beta):
    batch, seq = token_ids.shape
    vocab, hidden = word_emb.shape
    types = type_emb.shape[0]
    n_tokens = batch * seq

    tok = token_ids.reshape(n_tokens).astype(jnp.int32)
    tt = token_type_ids.reshape(n_tokens).astype(jnp.int32)

    sc_call = _make_sc_call(n_tokens, seq, vocab, types)
    out = sc_call(tok, tt, word_emb, pos_emb, type_emb, gamma, beta)
    return out.reshape(batch, seq, hidden)


# speedup vs baseline: 2.8657x; 2.8657x over previous
import functools

import jax
import jax.numpy as jnp
from jax import lax
from jax.experimental import pallas as pl
from jax.experimental.pallas import tpu as pltpu
from jax.experimental.pallas import tpu_sc as plsc

H = 768
NW = 32
CHUNK = 64
TBLK = 256


def _make_sc_gather(n_tokens):
    per_w = n_tokens // NW
    n_chunks = per_w // CHUNK
    assert n_chunks % 2 == 0
    mesh = plsc.VectorSubcoreMesh(core_axis_name="c", subcore_axis_name="s")

    @functools.partial(
        pl.kernel,
        out_type=jax.ShapeDtypeStruct((n_tokens, H), jnp.float32),
        mesh=mesh,
        compiler_params=pltpu.CompilerParams(
            use_tc_tiling_on_sc=False, needs_layout_passes=False),
        scratch_types=[
            pltpu.VMEM((per_w,), jnp.int32),
            pltpu.VMEM((CHUNK, H), jnp.float32),
            pltpu.VMEM((CHUNK, H), jnp.float32),
            pltpu.SemaphoreType.DMA,
            pltpu.SemaphoreType.DMA,
            pltpu.SemaphoreType.DMA,
            pltpu.SemaphoreType.DMA,
        ],
    )
    def sc_gather(tok_hbm, wemb_hbm, out_hbm, idx_v, buf0, buf1,
                  si0, si1, so0, so1):
        wid = lax.axis_index("s") * 2 + lax.axis_index("c")
        base = wid * per_w
        pltpu.sync_copy(tok_hbm.at[pl.ds(base, per_w)], idx_v)

        bufs = (buf0, buf1)
        sin = (si0, si1)
        sout = (so0, so1)

        def gather_in(c):
            return pltpu.async_copy(
                wemb_hbm.at[idx_v.at[pl.ds(c * CHUNK, CHUNK)]],
                bufs[c % 2], sin[c % 2])

        def copy_out(c):
            return pltpu.async_copy(
                bufs[c % 2], out_hbm.at[pl.ds(base + c * CHUNK, CHUNK)],
                sout[c % 2])

        ins = [gather_in(0), gather_in(1)]
        outs = [None, None]
        for c in range(n_chunks):
            ins[c % 2].wait()
            outs[c % 2] = copy_out(c)
            if c + 2 < n_chunks:
                outs[c % 2].wait()
                ins[c % 2] = gather_in(c + 2)
        outs[(n_chunks - 2) % 2].wait()
        outs[(n_chunks - 1) % 2].wait()

    return sc_gather


def _ln_body(gref, ttref, pref, tyref, gam, bet, oref):
    x = gref[...] + pref[...]
    ttf = ttref[0, 0, :].astype(jnp.float32)
    ty0 = tyref[0, :]
    dty = tyref[1, :] - ty0
    x = x + ty0[None, :] + ttf[:, None] * dty[None, :]
    mean = jnp.mean(x, axis=-1, keepdims=True)
    var = jnp.mean(x * x, axis=-1, keepdims=True) - mean * mean
    inv = lax.rsqrt(var + 1e-12)
    oref[...] = (x - mean) * inv * gam[...] + bet[...]


def _make_tc_ln(n_tokens, seq, types):
    n_blk = n_tokens // TBLK
    pos_blocks = seq // TBLK
    return pl.pallas_call(
        _ln_body,
        grid=(n_blk,),
        in_specs=[
            pl.BlockSpec((TBLK, H), lambda i: (i, 0)),
            pl.BlockSpec((1, 1, TBLK), lambda i: (i, 0, 0)),
            pl.BlockSpec((TBLK, H), lambda i: (lax.rem(i, pos_blocks), 0)),
            pl.BlockSpec((types, H), lambda i: (0, 0)),
            pl.BlockSpec((1, H), lambda i: (0, 0)),
            pl.BlockSpec((1, H), lambda i: (0, 0)),
        ],
        out_specs=pl.BlockSpec((TBLK, H), lambda i: (i, 0)),
        out_shape=jax.ShapeDtypeStruct((n_tokens, H), jnp.float32),
    )


def kernel(token_ids, token_type_ids, word_emb, pos_emb, type_emb, gamma, beta):
    batch, seq = token_ids.shape
    vocab, hidden = word_emb.shape
    types = type_emb.shape[0]
    n_tokens = batch * seq

    tok = token_ids.reshape(n_tokens).astype(jnp.int32)
    tt3 = token_type_ids.reshape(n_tokens // TBLK, 1, TBLK).astype(jnp.int32)

    gathered = _make_sc_gather(n_tokens)(tok, word_emb)
    out = _make_tc_ln(n_tokens, seq, types)(
        gathered, tt3, pos_emb, type_emb,
        gamma.reshape(1, hidden), beta.reshape(1, hidden))
    return out.reshape(batch, seq, hidden)


# speedup vs baseline: 14.6691x; 5.1189x over previous
import functools

import jax
import jax.numpy as jnp
from jax import lax
from jax.experimental import pallas as pl
from jax.experimental.pallas import tpu as pltpu
from jax.experimental.pallas import tpu_sc as plsc

H = 768
NW = 32
CHUNK = 64
TBLK = 256


def _make_sc_gather(n_tokens):
    per_w = n_tokens // NW
    n_chunks = per_w // CHUNK
    assert n_chunks % 2 == 0
    mesh = plsc.VectorSubcoreMesh(core_axis_name="c", subcore_axis_name="s")

    @functools.partial(
        pl.kernel,
        out_type=jax.ShapeDtypeStruct((n_tokens, H), jnp.float32),
        mesh=mesh,
        compiler_params=pltpu.CompilerParams(
            use_tc_tiling_on_sc=True, needs_layout_passes=False),
        scratch_types=[
            pltpu.VMEM((per_w,), jnp.int32),
            pltpu.VMEM((CHUNK, H), jnp.float32),
            pltpu.VMEM((CHUNK, H), jnp.float32),
            pltpu.SemaphoreType.DMA,
            pltpu.SemaphoreType.DMA,
            pltpu.SemaphoreType.DMA,
            pltpu.SemaphoreType.DMA,
        ],
    )
    def sc_gather(tok_hbm, wemb_hbm, out_hbm, idx_v, buf0, buf1,
                  si0, si1, so0, so1):
        wid = lax.axis_index("s") * 2 + lax.axis_index("c")
        base = wid * per_w
        pltpu.sync_copy(tok_hbm.at[pl.ds(base, per_w)], idx_v)

        bufs = (buf0, buf1)
        sin = (si0, si1)
        sout = (so0, so1)

        def gather_in(c):
            return pltpu.async_copy(
                wemb_hbm.at[idx_v.at[pl.ds(c * CHUNK, CHUNK)]],
                bufs[c % 2], sin[c % 2])

        def copy_out(c):
            return pltpu.async_copy(
                bufs[c % 2], out_hbm.at[pl.ds(base + c * CHUNK, CHUNK)],
                sout[c % 2])

        ins = [gather_in(0), gather_in(1)]
        outs = [None, None]
        for c in range(n_chunks):
            ins[c % 2].wait()
            outs[c % 2] = copy_out(c)
            if c + 2 < n_chunks:
                outs[c % 2].wait()
                ins[c % 2] = gather_in(c + 2)
        outs[(n_chunks - 2) % 2].wait()
        outs[(n_chunks - 1) % 2].wait()

    return sc_gather


def _ln_body(gref, ttref, pref, tyref, gam, bet, oref):
    x = gref[...] + pref[...]
    ttf = ttref[0, 0, :].astype(jnp.float32)
    ty0 = tyref[0, :]
    dty = tyref[1, :] - ty0
    x = x + ty0[None, :] + ttf[:, None] * dty[None, :]
    mean = jnp.mean(x, axis=-1, keepdims=True)
    var = jnp.mean(x * x, axis=-1, keepdims=True) - mean * mean
    inv = lax.rsqrt(var + 1e-12)
    oref[...] = (x - mean) * inv * gam[...] + bet[...]


def _make_tc_ln(n_tokens, seq, types):
    n_blk = n_tokens // TBLK
    pos_blocks = seq // TBLK
    return pl.pallas_call(
        _ln_body,
        grid=(n_blk,),
        in_specs=[
            pl.BlockSpec((TBLK, H), lambda i: (i, 0)),
            pl.BlockSpec((1, 1, TBLK), lambda i: (i, 0, 0)),
            pl.BlockSpec((TBLK, H), lambda i: (lax.rem(i, pos_blocks), 0)),
            pl.BlockSpec((types, H), lambda i: (0, 0)),
            pl.BlockSpec((1, H), lambda i: (0, 0)),
            pl.BlockSpec((1, H), lambda i: (0, 0)),
        ],
        out_specs=pl.BlockSpec((TBLK, H), lambda i: (i, 0)),
        out_shape=jax.ShapeDtypeStruct((n_tokens, H), jnp.float32),
    )


def kernel(token_ids, token_type_ids, word_emb, pos_emb, type_emb, gamma, beta):
    batch, seq = token_ids.shape
    vocab, hidden = word_emb.shape
    types = type_emb.shape[0]
    n_tokens = batch * seq

    tok = token_ids.reshape(n_tokens).astype(jnp.int32)
    tt3 = token_type_ids.reshape(n_tokens // TBLK, 1, TBLK).astype(jnp.int32)

    gathered = _make_sc_gather(n_tokens)(tok, word_emb)
    out = _make_tc_ln(n_tokens, seq, types)(
        gathered, tt3, pos_emb, type_emb,
        gamma.reshape(1, hidden), beta.reshape(1, hidden))
    return out.reshape(batch, seq, hidden)


# speedup vs baseline: 14.8469x; 1.0121x over previous
import functools

import jax
import jax.numpy as jnp
from jax import lax
from jax.experimental import pallas as pl
from jax.experimental.pallas import tpu as pltpu
from jax.experimental.pallas import tpu_sc as plsc

H = 768
NW = 32
CHUNK = 64
TBLK = 256


def _make_sc_gather(n_tokens):
    per_w = n_tokens // NW
    n_chunks = per_w // CHUNK
    assert n_chunks % 2 == 0
    mesh = plsc.VectorSubcoreMesh(core_axis_name="c", subcore_axis_name="s")

    @functools.partial(
        pl.kernel,
        out_type=jax.ShapeDtypeStruct((n_tokens, H), jnp.float32),
        mesh=mesh,
        compiler_params=pltpu.CompilerParams(
            use_tc_tiling_on_sc=True, needs_layout_passes=False),
        scratch_types=[
            pltpu.VMEM((per_w,), jnp.int32),
            pltpu.VMEM((CHUNK, H), jnp.float32),
            pltpu.VMEM((CHUNK, H), jnp.float32),
            pltpu.SemaphoreType.DMA,
            pltpu.SemaphoreType.DMA,
            pltpu.SemaphoreType.DMA,
            pltpu.SemaphoreType.DMA,
        ],
    )
    def sc_gather(tok_hbm, wemb_hbm, out_hbm, idx_v, buf0, buf1,
                  si0, si1, so0, so1):
        wid = lax.axis_index("s") * 2 + lax.axis_index("c")
        base = wid * per_w
        pltpu.sync_copy(tok_hbm.at[pl.ds(base, per_w)], idx_v)

        bufs = (buf0, buf1)
        sin = (si0, si1)
        sout = (so0, so1)

        def gather_in(c):
            return pltpu.async_copy(
                wemb_hbm.at[idx_v.at[pl.ds(c * CHUNK, CHUNK)]],
                bufs[c % 2], sin[c % 2])

        def copy_out(c):
            return pltpu.async_copy(
                bufs[c % 2], out_hbm.at[pl.ds(base + c * CHUNK, CHUNK)],
                sout[c % 2])

        ins = [gather_in(0), gather_in(1)]
        outs = [None, None]
        for c in range(n_chunks):
            ins[c % 2].wait()
            outs[c % 2] = copy_out(c)
            if c + 2 < n_chunks:
                outs[c % 2].wait()
                ins[c % 2] = gather_in(c + 2)
        outs[(n_chunks - 2) % 2].wait()
        outs[(n_chunks - 1) % 2].wait()

    return sc_gather


def _ln_body(gref, ttref, pref, tyref, gam, bet, oref):
    x = gref[...] + pref[...]
    ttf = ttref[0, 0, :].astype(jnp.float32)
    ty0 = tyref[0, :]
    dty = tyref[1, :] - ty0
    x = x + ty0[None, :] + ttf[:, None] * dty[None, :]
    mean = jnp.mean(x, axis=-1, keepdims=True)
    var = jnp.mean(x * x, axis=-1, keepdims=True) - mean * mean
    inv = lax.rsqrt(var + 1e-12)
    oref[...] = (x - mean) * inv * gam[...] + bet[...]


def _make_tc_ln(n_tokens, seq, types):
    batch = n_tokens // seq
    pos_blocks = seq // TBLK
    return pl.pallas_call(
        _ln_body,
        grid=(pos_blocks, batch),
        in_specs=[
            pl.BlockSpec((TBLK, H), lambda p, b: (b * pos_blocks + p, 0)),
            pl.BlockSpec((1, 1, TBLK), lambda p, b: (b * pos_blocks + p, 0, 0)),
            pl.BlockSpec((TBLK, H), lambda p, b: (p, 0)),
            pl.BlockSpec((types, H), lambda p, b: (0, 0)),
            pl.BlockSpec((1, H), lambda p, b: (0, 0)),
            pl.BlockSpec((1, H), lambda p, b: (0, 0)),
        ],
        out_specs=pl.BlockSpec((TBLK, H), lambda p, b: (b * pos_blocks + p, 0)),
        out_shape=jax.ShapeDtypeStruct((n_tokens, H), jnp.float32),
    )


def kernel(token_ids, token_type_ids, word_emb, pos_emb, type_emb, gamma, beta):
    batch, seq = token_ids.shape
    vocab, hidden = word_emb.shape
    types = type_emb.shape[0]
    n_tokens = batch * seq

    tok = token_ids.reshape(n_tokens).astype(jnp.int32)
    tt3 = token_type_ids.reshape(n_tokens // TBLK, 1, TBLK).astype(jnp.int32)

    gathered = _make_sc_gather(n_tokens)(tok, word_emb)
    out = _make_tc_ln(n_tokens, seq, types)(
        gathered, tt3, pos_emb, type_emb,
        gamma.reshape(1, hidden), beta.reshape(1, hidden))
    return out.reshape(batch, seq, hidden)


# speedup vs baseline: 17.1371x; 1.1542x over previous
import functools

import jax
import jax.numpy as jnp
from jax import lax
from jax.experimental import pallas as pl
from jax.experimental.pallas import tpu as pltpu
from jax.experimental.pallas import tpu_sc as plsc

H = 768
NW = 32
CHUNK = 64
TBLK = 512


def _make_sc_gather(n_tokens):
    per_w = n_tokens // NW
    n_chunks = per_w // CHUNK
    assert n_chunks % 2 == 0
    mesh = plsc.VectorSubcoreMesh(core_axis_name="c", subcore_axis_name="s")

    @functools.partial(
        pl.kernel,
        out_type=jax.ShapeDtypeStruct((n_tokens, H), jnp.float32),
        mesh=mesh,
        compiler_params=pltpu.CompilerParams(
            use_tc_tiling_on_sc=True, needs_layout_passes=False),
        scratch_types=[
            pltpu.VMEM((per_w,), jnp.int32),
            pltpu.VMEM((CHUNK, H), jnp.float32),
            pltpu.VMEM((CHUNK, H), jnp.float32),
            pltpu.SemaphoreType.DMA,
            pltpu.SemaphoreType.DMA,
            pltpu.SemaphoreType.DMA,
            pltpu.SemaphoreType.DMA,
        ],
    )
    def sc_gather(tok_hbm, wemb_hbm, out_hbm, idx_v, buf0, buf1,
                  si0, si1, so0, so1):
        wid = lax.axis_index("s") * 2 + lax.axis_index("c")
        base = wid * per_w
        pltpu.sync_copy(tok_hbm.at[pl.ds(base, per_w)], idx_v)

        bufs = (buf0, buf1)
        sin = (si0, si1)
        sout = (so0, so1)

        def gather_in(c):
            return pltpu.async_copy(
                wemb_hbm.at[idx_v.at[pl.ds(c * CHUNK, CHUNK)]],
                bufs[c % 2], sin[c % 2])

        def copy_out(c):
            return pltpu.async_copy(
                bufs[c % 2], out_hbm.at[pl.ds(base + c * CHUNK, CHUNK)],
                sout[c % 2])

        ins = [gather_in(0), gather_in(1)]
        outs = [None, None]
        for c in range(n_chunks):
            ins[c % 2].wait()
            outs[c % 2] = copy_out(c)
            if c + 2 < n_chunks:
                outs[c % 2].wait()
                ins[c % 2] = gather_in(c + 2)
        outs[(n_chunks - 2) % 2].wait()
        outs[(n_chunks - 1) % 2].wait()

    return sc_gather


def _ln_body(gref, ttref, pref, tyref, gam, bet, oref):
    x = gref[...] + pref[...]
    ttf = ttref[0, 0, :].astype(jnp.float32)
    ty0 = tyref[0, :]
    dty = tyref[1, :] - ty0
    x = x + ty0[None, :] + ttf[:, None] * dty[None, :]
    mean = jnp.mean(x, axis=-1, keepdims=True)
    var = jnp.mean(x * x, axis=-1, keepdims=True) - mean * mean
    inv = lax.rsqrt(var + 1e-12)
    oref[...] = (x - mean) * inv * gam[...] + bet[...]


def _make_tc_ln(n_tokens, seq, types):
    batch = n_tokens // seq
    pos_blocks = seq // TBLK
    return pl.pallas_call(
        _ln_body,
        grid=(pos_blocks, batch),
        in_specs=[
            pl.BlockSpec((TBLK, H), lambda p, b: (b * pos_blocks + p, 0)),
            pl.BlockSpec((1, 1, TBLK), lambda p, b: (b * pos_blocks + p, 0, 0)),
            pl.BlockSpec((TBLK, H), lambda p, b: (p, 0)),
            pl.BlockSpec((types, H), lambda p, b: (0, 0)),
            pl.BlockSpec((1, H), lambda p, b: (0, 0)),
            pl.BlockSpec((1, H), lambda p, b: (0, 0)),
        ],
        out_specs=pl.BlockSpec((TBLK, H), lambda p, b: (b * pos_blocks + p, 0)),
        out_shape=jax.ShapeDtypeStruct((n_tokens, H), jnp.float32),
    )


def kernel(token_ids, token_type_ids, word_emb, pos_emb, type_emb, gamma, beta):
    batch, seq = token_ids.shape
    vocab, hidden = word_emb.shape
    types = type_emb.shape[0]
    n_tokens = batch * seq

    tok = token_ids.reshape(n_tokens).astype(jnp.int32)
    tt3 = token_type_ids.reshape(n_tokens // TBLK, 1, TBLK).astype(jnp.int32)

    gathered = _make_sc_gather(n_tokens)(tok, word_emb)
    out = _make_tc_ln(n_tokens, seq, types)(
        gathered, tt3, pos_emb, type_emb,
        gamma.reshape(1, hidden), beta.reshape(1, hidden))
    return out.reshape(batch, seq, hidden)


# speedup vs baseline: 18.2201x; 1.0632x over previous
import functools

import jax
import jax.numpy as jnp
from jax import lax
from jax.experimental import pallas as pl
from jax.experimental.pallas import tpu as pltpu
from jax.experimental.pallas import tpu_sc as plsc

H = 768
NW = 32
CHUNK = 64
TBLK = 1024


def _make_sc_gather(n_tokens):
    per_w = n_tokens // NW
    n_chunks = per_w // CHUNK
    assert n_chunks % 2 == 0
    mesh = plsc.VectorSubcoreMesh(core_axis_name="c", subcore_axis_name="s")

    @functools.partial(
        pl.kernel,
        out_type=jax.ShapeDtypeStruct((n_tokens, H), jnp.float32),
        mesh=mesh,
        compiler_params=pltpu.CompilerParams(
            use_tc_tiling_on_sc=True, needs_layout_passes=False),
        scratch_types=[
            pltpu.VMEM((per_w,), jnp.int32),
            pltpu.VMEM((CHUNK, H), jnp.float32),
            pltpu.VMEM((CHUNK, H), jnp.float32),
            pltpu.SemaphoreType.DMA,
            pltpu.SemaphoreType.DMA,
            pltpu.SemaphoreType.DMA,
            pltpu.SemaphoreType.DMA,
        ],
    )
    def sc_gather(tok_hbm, wemb_hbm, out_hbm, idx_v, buf0, buf1,
                  si0, si1, so0, so1):
        wid = lax.axis_index("s") * 2 + lax.axis_index("c")
        base = wid * per_w
        pltpu.sync_copy(tok_hbm.at[pl.ds(base, per_w)], idx_v)

        bufs = (buf0, buf1)
        sin = (si0, si1)
        sout = (so0, so1)

        def gather_in(c):
            return pltpu.async_copy(
                wemb_hbm.at[idx_v.at[pl.ds(c * CHUNK, CHUNK)]],
                bufs[c % 2], sin[c % 2])

        def copy_out(c):
            return pltpu.async_copy(
                bufs[c % 2], out_hbm.at[pl.ds(base + c * CHUNK, CHUNK)],
                sout[c % 2])

        ins = [gather_in(0), gather_in(1)]
        outs = [None, None]
        for c in range(n_chunks):
            ins[c % 2].wait()
            outs[c % 2] = copy_out(c)
            if c + 2 < n_chunks:
                outs[c % 2].wait()
                ins[c % 2] = gather_in(c + 2)
        outs[(n_chunks - 2) % 2].wait()
        outs[(n_chunks - 1) % 2].wait()

    return sc_gather


def _ln_body(gref, ttref, pref, tyref, gam, bet, oref):
    x = gref[...] + pref[...]
    ttf = ttref[0, 0, :].astype(jnp.float32)
    ty0 = tyref[0, :]
    dty = tyref[1, :] - ty0
    x = x + ty0[None, :] + ttf[:, None] * dty[None, :]
    mean = jnp.mean(x, axis=-1, keepdims=True)
    var = jnp.mean(x * x, axis=-1, keepdims=True) - mean * mean
    inv = lax.rsqrt(var + 1e-12)
    oref[...] = (x - mean) * inv * gam[...] + bet[...]


def _make_tc_ln(n_tokens, seq, types):
    batch = n_tokens // seq
    pos_blocks = seq // TBLK
    return pl.pallas_call(
        _ln_body,
        grid=(pos_blocks, batch),
        in_specs=[
            pl.BlockSpec((TBLK, H), lambda p, b: (b * pos_blocks + p, 0)),
            pl.BlockSpec((1, 1, TBLK), lambda p, b: (b * pos_blocks + p, 0, 0)),
            pl.BlockSpec((TBLK, H), lambda p, b: (p, 0)),
            pl.BlockSpec((types, H), lambda p, b: (0, 0)),
            pl.BlockSpec((1, H), lambda p, b: (0, 0)),
            pl.BlockSpec((1, H), lambda p, b: (0, 0)),
        ],
        out_specs=pl.BlockSpec((TBLK, H), lambda p, b: (b * pos_blocks + p, 0)),
        out_shape=jax.ShapeDtypeStruct((n_tokens, H), jnp.float32),
    )


def kernel(token_ids, token_type_ids, word_emb, pos_emb, type_emb, gamma, beta):
    batch, seq = token_ids.shape
    vocab, hidden = word_emb.shape
    types = type_emb.shape[0]
    n_tokens = batch * seq

    tok = token_ids.reshape(n_tokens).astype(jnp.int32)
    tt3 = token_type_ids.reshape(n_tokens // TBLK, 1, TBLK).astype(jnp.int32)

    gathered = _make_sc_gather(n_tokens)(tok, word_emb)
    out = _make_tc_ln(n_tokens, seq, types)(
        gathered, tt3, pos_emb, type_emb,
        gamma.reshape(1, hidden), beta.reshape(1, hidden))
    return out.reshape(batch, seq, hidden)


# speedup vs baseline: 18.7371x; 1.0284x over previous
import functools

import jax
import jax.numpy as jnp
from jax import lax
from jax.experimental import pallas as pl
from jax.experimental.pallas import tpu as pltpu
from jax.experimental.pallas import tpu_sc as plsc

H = 768
NW = 32
CHUNK = 64
TBLK = 2048


def _make_sc_gather(n_tokens):
    per_w = n_tokens // NW
    n_chunks = per_w // CHUNK
    assert n_chunks % 2 == 0
    mesh = plsc.VectorSubcoreMesh(core_axis_name="c", subcore_axis_name="s")

    @functools.partial(
        pl.kernel,
        out_type=jax.ShapeDtypeStruct((n_tokens, H), jnp.float32),
        mesh=mesh,
        compiler_params=pltpu.CompilerParams(
            use_tc_tiling_on_sc=True, needs_layout_passes=False),
        scratch_types=[
            pltpu.VMEM((per_w,), jnp.int32),
            pltpu.VMEM((CHUNK, H), jnp.float32),
            pltpu.VMEM((CHUNK, H), jnp.float32),
            pltpu.SemaphoreType.DMA,
            pltpu.SemaphoreType.DMA,
            pltpu.SemaphoreType.DMA,
            pltpu.SemaphoreType.DMA,
        ],
    )
    def sc_gather(tok_hbm, wemb_hbm, out_hbm, idx_v, buf0, buf1,
                  si0, si1, so0, so1):
        wid = lax.axis_index("s") * 2 + lax.axis_index("c")
        base = wid * per_w
        pltpu.sync_copy(tok_hbm.at[pl.ds(base, per_w)], idx_v)

        bufs = (buf0, buf1)
        sin = (si0, si1)
        sout = (so0, so1)

        def gather_in(c):
            return pltpu.async_copy(
                wemb_hbm.at[idx_v.at[pl.ds(c * CHUNK, CHUNK)]],
                bufs[c % 2], sin[c % 2])

        def copy_out(c):
            return pltpu.async_copy(
                bufs[c % 2], out_hbm.at[pl.ds(base + c * CHUNK, CHUNK)],
                sout[c % 2])

        ins = [gather_in(0), gather_in(1)]
        outs = [None, None]
        for c in range(n_chunks):
            ins[c % 2].wait()
            outs[c % 2] = copy_out(c)
            if c + 2 < n_chunks:
                outs[c % 2].wait()
                ins[c % 2] = gather_in(c + 2)
        outs[(n_chunks - 2) % 2].wait()
        outs[(n_chunks - 1) % 2].wait()

    return sc_gather


def _ln_body(gref, ttref, pref, tyref, gam, bet, oref):
    x = gref[...] + pref[...]
    ttf = ttref[0, 0, :].astype(jnp.float32)
    ty0 = tyref[0, :]
    dty = tyref[1, :] - ty0
    x = x + ty0[None, :] + ttf[:, None] * dty[None, :]
    mean = jnp.mean(x, axis=-1, keepdims=True)
    var = jnp.mean(x * x, axis=-1, keepdims=True) - mean * mean
    inv = lax.rsqrt(var + 1e-12)
    oref[...] = (x - mean) * inv * gam[...] + bet[...]


def _make_tc_ln(n_tokens, seq, types):
    batch = n_tokens // seq
    pos_blocks = seq // TBLK
    return pl.pallas_call(
        _ln_body,
        grid=(pos_blocks, batch),
        in_specs=[
            pl.BlockSpec((TBLK, H), lambda p, b: (b * pos_blocks + p, 0)),
            pl.BlockSpec((1, 1, TBLK), lambda p, b: (b * pos_blocks + p, 0, 0)),
            pl.BlockSpec((TBLK, H), lambda p, b: (p, 0)),
            pl.BlockSpec((types, H), lambda p, b: (0, 0)),
            pl.BlockSpec((1, H), lambda p, b: (0, 0)),
            pl.BlockSpec((1, H), lambda p, b: (0, 0)),
        ],
        out_specs=pl.BlockSpec((TBLK, H), lambda p, b: (b * pos_blocks + p, 0)),
        out_shape=jax.ShapeDtypeStruct((n_tokens, H), jnp.float32),
    )


def kernel(token_ids, token_type_ids, word_emb, pos_emb, type_emb, gamma, beta):
    batch, seq = token_ids.shape
    vocab, hidden = word_emb.shape
    types = type_emb.shape[0]
    n_tokens = batch * seq

    tok = token_ids.reshape(n_tokens).astype(jnp.int32)
    tt3 = token_type_ids.reshape(n_tokens // TBLK, 1, TBLK).astype(jnp.int32)

    gathered = _make_sc_gather(n_tokens)(tok, word_emb)
    out = _make_tc_ln(n_tokens, seq, types)(
        gathered, tt3, pos_emb, type_emb,
        gamma.reshape(1, hidden), beta.reshape(1, hidden))
    return out.reshape(batch, seq, hidden)
